# Initial kernel scaffold; baseline (speedup 1.0000x reference)
import jax
import jax.numpy as jnp
from jax.experimental import pallas as pl


def kernel(x, edge_index, W_in, b_in, ln1_g, ln1_b, W1_w, W1_b, ln3_g, ln3_b, W3_w, W3_b, cls_w1, cls_b1, cls_lng, cls_lnb, cls_w2, cls_b2):
    raise NotImplementedError("write your pallas kernel here")


# speedup vs baseline: 6.3247x; 6.3247x over previous
import functools

import jax
import jax.numpy as jnp
from jax import lax
from jax.experimental import pallas as pl
from jax.experimental.pallas import tpu as pltpu
from jax.experimental.pallas import tpu_sc as plsc

N = 10000
NNZ = 320000
H = 128
NE = 10000
OUT = 40
ALPHA = 0.5


BN = 2000
GRID = N // BN


def _ln(x, g, b):
    mu = jnp.mean(x, axis=-1, keepdims=True)
    var = jnp.mean((x - mu) ** 2, axis=-1, keepdims=True)
    return (x - mu) / jnp.sqrt(var + 1e-5) * g + b


def _mm(a, w):
    return lax.dot_general(a, w, (((1,), (0,)), ((), ())),
                           preferred_element_type=jnp.float32)


def _rowspec():
    return pl.BlockSpec((1, H), lambda i: (0, 0))


def _matspec():
    return pl.BlockSpec((H, H), lambda i: (0, 0))


def _blkspec():
    return pl.BlockSpec((BN, H), lambda i: (i, 0))


def _pspec():
    return pl.BlockSpec((2, BN, H), lambda i: (0, i, 0))


def _pre_body(x_ref, win, bin_, g1, b1, w1, wb1, h0_ref, t_ref):
    h = jnp.maximum(_mm(x_ref[...], win[...]) + bin_[...], 0.0)
    h0_ref[...] = h
    t_ref[...] = _mm(_ln(h, g1[...], b1[...]), w1[...]) + wb1[...]


def _combine_body(xep_ref, xe_ref):
    xe_ref[...] = xep_ref[0] + xep_ref[1]


def _mid_body(xvp_ref, h0_ref, g3, b3, w3, wb3, g1, b1, w1, wb1, t_ref):
    xv = xvp_ref[0] + xvp_ref[1]
    hn = (1.0 - ALPHA) * xv + ALPHA * h0_ref[...]
    hn = _mm(_ln(hn, g3[...], b3[...]), w3[...]) + wb3[...]
    h = jnp.maximum(hn, 0.0)
    t_ref[...] = _mm(_ln(h, g1[...], b1[...]), w1[...]) + wb1[...]


def _final_body(xvp_ref, h0_ref, g3, b3, w3, wb3, cw1, cb1, clng, clnb,
                cw2, cb2, out_ref):
    xv = xvp_ref[0] + xvp_ref[1]
    hn = (1.0 - ALPHA) * xv + ALPHA * h0_ref[...]
    hn = _mm(_ln(hn, g3[...], b3[...]), w3[...]) + wb3[...]
    h = jnp.maximum(hn, 0.0)
    c = jnp.maximum(_mm(h, cw1[...]) + cb1[...], 0.0)
    c = _ln(c, clng[...], clnb[...])
    out_ref[...] = _mm(c, cw2[...]) + cb2[...]


def _tc_pre(x, W_in, b_in, ln1_g, ln1_b, W1_w, W1_b):
    return pl.pallas_call(
        _pre_body,
        grid=(GRID,),
        in_specs=[_blkspec(), _matspec(), _rowspec(), _rowspec(), _rowspec(),
                  _matspec(), _rowspec()],
        out_specs=[_blkspec(), _blkspec()],
        out_shape=[jax.ShapeDtypeStruct((N, H), jnp.float32)] * 2,
    )(x, W_in, b_in.reshape(1, H), ln1_g.reshape(1, H), ln1_b.reshape(1, H),
      W1_w, W1_b.reshape(1, H))


def _tc_combine(xep):
    return pl.pallas_call(
        _combine_body,
        grid=(GRID,),
        in_specs=[_pspec()],
        out_specs=_blkspec(),
        out_shape=jax.ShapeDtypeStruct((NE, H), jnp.float32),
    )(xep)


def _tc_mid(xvp, h0, ln3_g, ln3_b, W3_w, W3_b, ln1_g, ln1_b, W1_w, W1_b):
    return pl.pallas_call(
        _mid_body,
        grid=(GRID,),
        in_specs=[_pspec(), _blkspec(), _rowspec(), _rowspec(), _matspec(),
                  _rowspec(), _rowspec(), _rowspec(), _matspec(), _rowspec()],
        out_specs=_blkspec(),
        out_shape=jax.ShapeDtypeStruct((N, H), jnp.float32),
    )(xvp, h0, ln3_g.reshape(1, H), ln3_b.reshape(1, H), W3_w,
      W3_b.reshape(1, H), ln1_g.reshape(1, H), ln1_b.reshape(1, H), W1_w,
      W1_b.reshape(1, H))


def _tc_final(xvp, h0, ln3_g, ln3_b, W3_w, W3_b, cls_w1, cls_b1, cls_lng,
              cls_lnb, cls_w2p, cls_b2p):
    return pl.pallas_call(
        _final_body,
        grid=(GRID,),
        in_specs=[_pspec(), _blkspec(), _rowspec(), _rowspec(), _matspec(),
                  _rowspec(), _matspec(), _rowspec(), _rowspec(), _rowspec(),
                  _matspec(), _rowspec()],
        out_specs=_blkspec(),
        out_shape=jax.ShapeDtypeStruct((N, H), jnp.float32),
    )(xvp, h0, ln3_g.reshape(1, H), ln3_b.reshape(1, H), W3_w,
      W3_b.reshape(1, H), cls_w1, cls_b1.reshape(1, H),
      cls_lng.reshape(1, H), cls_lnb.reshape(1, H), cls_w2p,
      cls_b2p.reshape(1, H))


NW = 32
PT = NNZ // NW
CH = 80
NCH = PT // CH
SB = CH
NSB = NE // SB
MAXSB = -(-NSB // 16)

_sc_mesh = plsc.VectorSubcoreMesh(core_axis_name="c", subcore_axis_name="s")


@functools.partial(
    pl.kernel,
    mesh=_sc_mesh,
    out_type=jax.ShapeDtypeStruct((2, NE, H), jnp.float32),
    scratch_types=[
        pltpu.VMEM((NCH, CH), jnp.int32),
        pltpu.VMEM((NCH, CH), jnp.int32),
        pltpu.VMEM((CH, H), jnp.float32),
        pltpu.VMEM_SHARED((NE, H), jnp.float32),
        pltpu.SemaphoreType.DMA,
    ],
)
def _sc_segsum(src, gidx, sidx, out, idx_g, idx_s, rows, acc, sem):
    c = lax.axis_index("c")
    s = lax.axis_index("s")
    wid = s * 2 + c
    pltpu.sync_copy(gidx.at[wid], idx_g)
    pltpu.sync_copy(sidx.at[wid], idx_s)

    def _zb(i, carry):
        rows[i // 8, pl.ds((i % 8) * 16, 16)] = jnp.zeros((16,), jnp.float32)
        return carry
    lax.fori_loop(0, SB * 8, _zb, 0)
    for i in range(MAXSB):
        k = s + 16 * i

        @pl.when(k < NSB)
        def _():
            pltpu.sync_copy(rows, acc.at[pl.ds(k * SB, SB)])
    plsc.subcore_barrier()

    def _chunk(g, carry):
        pltpu.async_copy(src.at[idx_g.at[g]], rows, sem).wait()
        pltpu.sync_copy(rows, acc.at[idx_s.at[g]], add=True)
        return carry
    lax.fori_loop(0, NCH, _chunk, 0)
    plsc.subcore_barrier()

    for i in range(MAXSB):
        k = s + 16 * i

        @pl.when(k < NSB)
        def _():
            pltpu.sync_copy(acc.at[pl.ds(k * SB, SB)], rows)
            pltpu.sync_copy(rows, out.at[c, pl.ds(k * SB, SB)])


def kernel(x, edge_index, W_in, b_in, ln1_g, ln1_b, W1_w, W1_b, ln3_g, ln3_b,
           W3_w, W3_b, cls_w1, cls_b1, cls_lng, cls_lnb, cls_w2, cls_b2):
    vtx = edge_index[0].reshape(NW, NCH, CH)
    edg = edge_index[1].reshape(NW, NCH, CH)
    cls_w2p = jnp.pad(cls_w2, ((0, 0), (0, H - OUT)))
    cls_b2p = jnp.pad(cls_b2, (0, H - OUT))

    h0, t = _tc_pre(x, W_in, b_in, ln1_g, ln1_b, W1_w, W1_b)

    xep = _sc_segsum(t, vtx, edg)
    xe = _tc_combine(xep)
    xvp = _sc_segsum(xe, edg, vtx)
    t = _tc_mid(xvp, h0, ln3_g, ln3_b, W3_w, W3_b, ln1_g, ln1_b, W1_w, W1_b)

    xep = _sc_segsum(t, vtx, edg)
    xe = _tc_combine(xep)
    xvp = _sc_segsum(xe, edg, vtx)
    outp = _tc_final(xvp, h0, ln3_g, ln3_b, W3_w, W3_b, cls_w1, cls_b1,
                     cls_lng, cls_lnb, cls_w2p, cls_b2p)
    return outp[:, :OUT]
